# baseline (speedup 1.0000x reference)
import functools
import math

import jax
import jax.numpy as jnp
from jax import lax
from jax.experimental import pallas as pl
from jax.experimental.pallas import tpu as pltpu
from jax.experimental.pallas import tpu_sc as plsc

DEPTH = 5
RATIO = 0.5
NUM_SC = 2
NUM_TILES = 16
NUM_W = NUM_SC * NUM_TILES
ROW_ALIGN = 512


def _round_up(a, b):
    return (a + b - 1) // b * b


def _sc_mesh():
    return plsc.VectorSubcoreMesh(core_axis_name="c", subcore_axis_name="s",
                                  num_cores=NUM_SC, num_subcores=NUM_TILES)


_SC_PARAMS = pltpu.CompilerParams(use_tc_tiling_on_sc=False)


def _mm_body(a_ref, w_ref, deg_ref, o_ref, *, n, bm):
    acc = jnp.dot(a_ref[...], w_ref[...], preferred_element_type=jnp.float32)
    rs = 1.0 / jnp.sqrt(deg_ref[...] + 2.0)
    i = pl.program_id(0)
    rows = lax.broadcasted_iota(jnp.int32, acc.shape, 0) + i * bm
    o_ref[...] = jnp.where(rows < n, acc * rs, 0.0)


def _mm_prescale(a, w, deg_col, n, bm=ROW_ALIGN):
    mpad, k = a.shape
    f = w.shape[1]
    return pl.pallas_call(
        functools.partial(_mm_body, n=n, bm=bm),
        grid=(mpad // bm,),
        in_specs=[
            pl.BlockSpec((bm, k), lambda i: (i, 0)),
            pl.BlockSpec((k, f), lambda i: (0, 0)),
            pl.BlockSpec((bm, 1), lambda i: (i, 0)),
        ],
        out_specs=pl.BlockSpec((bm, f), lambda i: (i, 0)),
        out_shape=jax.ShapeDtypeStruct((mpad, f), jnp.float32),
    )(a, w, deg_col)


def _combine_body(acc_ref, hp_ref, deg_ref, b_ref, o_ref, *, relu):
    s = acc_ref[...] + 2.0 * hp_ref[...]
    rs = 1.0 / jnp.sqrt(deg_ref[...] + 2.0)
    r = s * rs + b_ref[...]
    if relu:
        r = jnp.maximum(r, 0.0)
    o_ref[...] = r


def _combine(acc, hp, deg_col, b, n, relu, bm=ROW_ALIGN):
    f = hp.shape[1]
    return pl.pallas_call(
        functools.partial(_combine_body, relu=relu),
        grid=(_round_up(n, bm) // bm,),
        in_specs=[
            pl.BlockSpec((bm, f), lambda i: (i, 0)),
            pl.BlockSpec((bm, f), lambda i: (i, 0)),
            pl.BlockSpec((bm, 1), lambda i: (i, 0)),
            pl.BlockSpec((1, f), lambda i: (0, 0)),
        ],
        out_specs=pl.BlockSpec((bm, f), lambda i: (i, 0)),
        out_shape=jax.ShapeDtypeStruct((n, f), jnp.float32),
    )(acc, hp, deg_col, b.reshape(1, f))


def _score_body(x_ref, w_ref, o_ref, y_ref, *, n, bm):
    wv = w_ref[...]
    nrm = jnp.sqrt(jnp.sum(wv * wv))
    xv = x_ref[...]
    s = jnp.dot(xv, wv, preferred_element_type=jnp.float32) / nrm
    t = jnp.tanh(s)
    i = pl.program_id(0)
    rows = lax.broadcasted_iota(jnp.int32, t.shape, 0) + i * bm
    o_ref[...] = jnp.where(rows < n, t, -2.0)
    y_ref[...] = xv * t


def _score(x_pad, w, n, bm=ROW_ALIGN):
    npad, c = x_pad.shape
    return pl.pallas_call(
        functools.partial(_score_body, n=n, bm=bm),
        grid=(npad // bm,),
        in_specs=[pl.BlockSpec((bm, c), lambda i: (i, 0)),
                  pl.BlockSpec((c, 1), lambda i: (0, 0))],
        out_specs=[pl.BlockSpec((bm, 1), lambda i: (i, 0)),
                   pl.BlockSpec((bm, c), lambda i: (i, 0))],
        out_shape=[jax.ShapeDtypeStruct((npad, 1), jnp.float32),
                   jax.ShapeDtypeStruct((npad, c), jnp.float32)],
    )(x_pad, w.reshape(c, 1))


def _orderable_i32(b):
    return jnp.where(b < 0, ~b, b | jnp.int32(-2147483648))


def _thresh_body(s_ref, m_ref, *, k):
    b = lax.bitcast_convert_type(s_ref[...], jnp.int32)
    keys = _orderable_i32(b).astype(jnp.uint32)
    acc = jnp.uint32(0)
    for bitpos in range(31, -1, -1):
        cand = acc | jnp.uint32(1 << bitpos)
        cnt = jnp.sum((keys >= cand).astype(jnp.int32))
        acc = jnp.where(cnt >= k, cand, acc)
    m_ref[...] = (keys >= acc).astype(jnp.int32)


def _thresh(score_col, k):
    npad = score_col.shape[0]
    return pl.pallas_call(
        functools.partial(_thresh_body, k=k),
        in_specs=[pl.BlockSpec((npad, 1), lambda: (0, 0))],
        out_specs=pl.BlockSpec((npad, 1), lambda: (0, 0)),
        out_shape=jax.ShapeDtypeStruct((npad, 1), jnp.int32),
    )(score_col)


def _gather_body(table, idx, out, idxv, rows, sem, *, bg, nbat, trash):
    cid = lax.axis_index("c")
    sid = lax.axis_index("s")
    wid = cid + NUM_SC * sid

    def body(i, _):
        base = (wid + i * NUM_W) * bg
        pltpu.sync_copy(idx.at[pl.ds(base, bg)], idxv)
        for v in range(bg // 16):
            sl = pl.ds(v * 16, 16)
            m = idxv[sl]
            idxv[sl] = jnp.where(m < 0, trash, m)
        pltpu.async_copy(table.at[idxv], rows, sem).wait()
        pltpu.sync_copy(rows, out.at[pl.ds(base, bg)])
        return 0

    lax.fori_loop(0, nbat, body, 0)


def _gather_rows(table, idx, bg=16):
    nout = idx.shape[0]
    npad_in, f = table.shape
    nbat = nout // (NUM_W * bg)
    body = functools.partial(_gather_body, bg=bg, nbat=nbat,
                             trash=npad_in - 1)
    return pl.kernel(
        body,
        out_type=jax.ShapeDtypeStruct((nout, f), jnp.float32),
        mesh=_sc_mesh(),
        compiler_params=_SC_PARAMS,
        scratch_types=[
            pltpu.VMEM((bg,), jnp.int32),
            pltpu.VMEM((bg, f), jnp.float32),
            pltpu.SemaphoreType.DMA,
        ],
    )(table, idx)


def _scatter_body(ysrc, idx, out, idxv, rows, *, bg, nbat, trash):
    cid = lax.axis_index("c")
    sid = lax.axis_index("s")
    wid = cid + NUM_SC * sid

    def body(i, _):
        base = (wid + i * NUM_W) * bg
        pltpu.sync_copy(idx.at[pl.ds(base, bg)], idxv)
        for v in range(bg // 16):
            sl = pl.ds(v * 16, 16)
            m = idxv[sl]
            idxv[sl] = jnp.where(m < 0, trash, m)
        pltpu.sync_copy(ysrc.at[pl.ds(base, bg)], rows)
        pltpu.sync_copy(rows, out.at[idxv])
        return 0

    lax.fori_loop(0, nbat, body, 0)


def _scatter_rows(ysrc, idx, npad_out, bg=16):
    npad_in, f = ysrc.shape
    nbat = npad_in // (NUM_W * bg)
    body = functools.partial(_scatter_body, bg=bg, nbat=nbat,
                             trash=npad_out - 1)
    return pl.kernel(
        body,
        out_type=jax.ShapeDtypeStruct((npad_out, f), jnp.float32),
        mesh=_sc_mesh(),
        compiler_params=_SC_PARAMS,
        scratch_types=[
            pltpu.VMEM((bg,), jnp.int32),
            pltpu.VMEM((bg, f), jnp.float32),
        ],
    )(ysrc, idx)


def _agg_body(hp, srcp, dstp, nbv, out, idx_s, idx_d, rows, nb_v, sem,
              acc_sp, *, npad, f, be, ecap):
    cid = lax.axis_index("c")
    sid = lax.axis_index("s")
    half = npad // NUM_SC

    def zrow(r, _):
        for c in range(f // 16):
            rows[r, pl.ds(c * 16, 16)] = jnp.zeros((16,), jnp.float32)
        return 0

    lax.fori_loop(0, 16, zrow, 0)
    rpt = half // NUM_TILES

    def zcp(j, _):
        pltpu.sync_copy(rows.at[pl.ds(0, 16)],
                        acc_sp.at[pl.ds(sid * rpt + j * 16, 16)])
        return 0

    lax.fori_loop(0, rpt // 16, zcp, 0)
    plsc.subcore_barrier()

    pltpu.sync_copy(nbv, nb_v)
    nbs = nb_v[...]
    nb = jnp.where(cid == 0, nbs[0], nbs[1])
    nmine = (nb - sid + NUM_TILES - 1) // NUM_TILES
    rebase = cid * half

    def body(i, _):
        base = cid * ecap + (sid + i * NUM_TILES) * be
        pltpu.sync_copy(srcp.at[pl.ds(base, be)], idx_s)
        pltpu.sync_copy(dstp.at[pl.ds(base, be)], idx_d)
        for v in range(be // 16):
            sl = pl.ds(v * 16, 16)
            idx_d[sl] = idx_d[sl] - rebase
        pltpu.async_copy(hp.at[idx_s], rows, sem).wait()
        pltpu.sync_copy(rows, acc_sp.at[idx_d], add=True)
        return 0

    lax.fori_loop(0, nmine, body, 0)
    plsc.subcore_barrier()

    def dump(j, _):
        r0 = sid * rpt + j * 16
        pltpu.sync_copy(acc_sp.at[pl.ds(r0, 16)], rows.at[pl.ds(0, 16)])
        pltpu.sync_copy(rows.at[pl.ds(0, 16)], out.at[cid, pl.ds(r0, 16)])
        return 0

    lax.fori_loop(0, rpt // 16, dump, 0)


def _agg(hp, srcp, dstp, nbv, npad, f, be):
    ecap = srcp.shape[0] // NUM_SC
    body = functools.partial(_agg_body, npad=npad, f=f, be=be, ecap=ecap)
    out = pl.kernel(
        body,
        out_type=jax.ShapeDtypeStruct((NUM_SC, npad // NUM_SC, f),
                                      jnp.float32),
        mesh=_sc_mesh(),
        compiler_params=_SC_PARAMS,
        scratch_types=[
            pltpu.VMEM((be,), jnp.int32),
            pltpu.VMEM((be,), jnp.int32),
            pltpu.VMEM((be, f), jnp.float32),
            pltpu.VMEM((16,), jnp.int32),
            pltpu.SemaphoreType.DMA,
            pltpu.VMEM_SHARED((npad // NUM_SC, f), jnp.float32),
        ],
    )(hp, srcp, dstp, nbv)
    return out.reshape(npad, f)


def _gcn(x_pad, w, b, deg_col, srcp, dstp, nbv, n, npad, be, relu):
    hp = _mm_prescale(x_pad, w, deg_col, n)
    acc = _agg(hp, srcp, dstp, nbv, npad, w.shape[1], be)
    return _combine(acc, hp, deg_col, b, n, relu)


def _be_for(f):
    return 64 if f >= 1024 else 128


def _pad_rows(a, npad):
    return jnp.pad(a, ((0, npad - a.shape[0]), (0, 0)))


def _bucket(src, dst, valid, e_cap, npad):
    half = npad // NUM_SC
    low = dst < half
    m0 = valid & low
    m1 = valid & ~low
    p0 = jnp.cumsum(m0.astype(jnp.int32)) - 1
    p1 = jnp.cumsum(m1.astype(jnp.int32)) - 1
    cnt0 = jnp.sum(m0.astype(jnp.int32))
    cnt1 = jnp.sum(m1.astype(jnp.int32))
    drop = 2 * e_cap
    tgt = jnp.where(m0, p0, jnp.where(m1, e_cap + p1, drop))
    bsrc = jnp.full((2 * e_cap,), npad - 1, jnp.int32).at[tgt].set(
        src, mode="drop")
    ar2 = jnp.arange(2 * e_cap, dtype=jnp.int32)
    fill = jnp.where(ar2 < e_cap, 0, half)
    bdst = fill.at[tgt].set(dst, mode="drop")
    return bsrc, bdst, cnt0, cnt1


def _nbv(cnt0, cnt1, be):
    return jnp.stack([(cnt0 + be - 1) // be, (cnt1 + be - 1) // be] +
                     [jnp.int32(0)] * 14).astype(jnp.int32)


def kernel(x, edge_index, Wd, bd, pw, Wu, bu):
    ns = [x.shape[0]]
    for _ in range(DEPTH):
        ns.append(int(math.ceil(RATIO * ns[-1])))
    npads = [_round_up(n + 1, ROW_ALIGN) for n in ns]
    e_cap = edge_index.shape[1]

    src = edge_index[0].astype(jnp.int32)
    dst = edge_index[1].astype(jnp.int32)

    def deg_of(bsrc, bdst, c0, c1, n, npad):
        onecol = _pad_rows(jnp.ones((n, 16), jnp.float32), npad)
        d16 = _agg(onecol, bsrc, bdst, _nbv(c0, c1, 128), npad, 16, 128)
        return d16[:, :1]

    n0, npad0 = ns[0], npads[0]
    bsrc, bdst, cnt0, cnt1 = _bucket(
        src, dst, jnp.ones((e_cap,), bool), e_cap, npad0)
    deg0_col = deg_of(bsrc, bdst, cnt0, cnt1, n0, npad0)
    be0 = _be_for(Wd[0].shape[1])
    xcur = _gcn(_pad_rows(x, npad0), Wd[0], bd[0], deg0_col,
                bsrc, bdst, _nbv(cnt0, cnt1, be0), n0, npad0, be0, relu=True)

    xs = [xcur]
    lvl = [(bsrc, bdst, cnt0, cnt1, deg0_col)]
    maps = []

    for i in range(1, DEPTH + 1):
        n_prev, n_i = ns[i - 1], ns[i]
        npad_prev, npad = npads[i - 1], npads[i]
        bsrc_p, bdst_p, c0_p, c1_p, _ = lvl[i - 1]

        x_prev_pad = _pad_rows(xcur, npad_prev)
        score_col, y = _score(x_prev_pad, pw[i - 1], n_prev)
        msel = _thresh(score_col, n_i).reshape(npad_prev)
        pos = jnp.cumsum(msel) - 1
        mapping = jnp.where((msel > 0) & (pos < n_i), pos, -1).astype(jnp.int32)

        rs = mapping[bsrc_p]
        rd = mapping[bdst_p]
        valid = (rs >= 0) & (rd >= 0)

        fo = Wd[i].shape[1]
        be = _be_for(fo)
        bsrc, bdst, cnt0, cnt1 = _bucket(rs, rd, valid, e_cap, npad)
        deg_col = deg_of(bsrc, bdst, cnt0, cnt1, n_i, npad)
        xn_pad = _scatter_rows(y, mapping, npad)
        xcur = _gcn(xn_pad, Wd[i], bd[i], deg_col,
                    bsrc, bdst, _nbv(cnt0, cnt1, be), n_i, npad, be,
                    relu=True)
        maps.append(mapping)
        if i < DEPTH:
            xs.append(xcur)
            lvl.append((bsrc, bdst, cnt0, cnt1, deg_col))

    for i in range(DEPTH):
        j = DEPTH - 1 - i
        n_j, npad_j = ns[j], npads[j]
        res = xs[j]
        xcur_pad = _pad_rows(xcur, npads[j + 1])
        up_pad = _gather_rows(xcur_pad, maps[j])
        cat = jnp.concatenate([_pad_rows(res, npad_j), up_pad], axis=-1)

        fo = Wu[i].shape[1]
        be = _be_for(fo)
        bsrc_j, bdst_j, c0_j, c1_j, deg_j = lvl[j]
        xcur = _gcn(cat, Wu[i], bu[i], deg_j, bsrc_j, bdst_j,
                    _nbv(c0_j, c1_j, be), n_j, npad_j, be,
                    relu=(i < DEPTH - 1))

    return xcur


# speedup vs baseline: 2.2638x; 2.2638x over previous
import functools
import math

import jax
import jax.numpy as jnp
from jax import lax
from jax.experimental import pallas as pl
from jax.experimental.pallas import tpu as pltpu
from jax.experimental.pallas import tpu_sc as plsc

DEPTH = 5
RATIO = 0.5
NUM_SC = 2
NUM_TILES = 16
NUM_W = NUM_SC * NUM_TILES
ROW_ALIGN = 512


def _round_up(a, b):
    return (a + b - 1) // b * b


def _sc_mesh():
    return plsc.VectorSubcoreMesh(core_axis_name="c", subcore_axis_name="s",
                                  num_cores=NUM_SC, num_subcores=NUM_TILES)


_SC_PARAMS = pltpu.CompilerParams(use_tc_tiling_on_sc=False)


def _mm_body(a_ref, w_ref, deg_ref, o_ref, *, n, bm):
    acc = jnp.dot(a_ref[...], w_ref[...], preferred_element_type=jnp.float32)
    rs = 1.0 / jnp.sqrt(deg_ref[...] + 2.0)
    i = pl.program_id(0)
    rows = lax.broadcasted_iota(jnp.int32, acc.shape, 0) + i * bm
    o_ref[...] = jnp.where(rows < n, acc * rs, 0.0)


def _mm_prescale(a, w, deg_col, n, bm=ROW_ALIGN):
    mpad, k = a.shape
    f = w.shape[1]
    return pl.pallas_call(
        functools.partial(_mm_body, n=n, bm=bm),
        grid=(mpad // bm,),
        in_specs=[
            pl.BlockSpec((bm, k), lambda i: (i, 0)),
            pl.BlockSpec((k, f), lambda i: (0, 0)),
            pl.BlockSpec((bm, 1), lambda i: (i, 0)),
        ],
        out_specs=pl.BlockSpec((bm, f), lambda i: (i, 0)),
        out_shape=jax.ShapeDtypeStruct((mpad, f), jnp.float32),
    )(a, w, deg_col)


def _combine_body(acc_ref, hp_ref, deg_ref, b_ref, o_ref, *, relu):
    s = acc_ref[...] + 2.0 * hp_ref[...]
    rs = 1.0 / jnp.sqrt(deg_ref[...] + 2.0)
    r = s * rs + b_ref[...]
    if relu:
        r = jnp.maximum(r, 0.0)
    o_ref[...] = r


def _combine(acc, hp, deg_col, b, n, relu, bm=ROW_ALIGN):
    f = hp.shape[1]
    return pl.pallas_call(
        functools.partial(_combine_body, relu=relu),
        grid=(_round_up(n, bm) // bm,),
        in_specs=[
            pl.BlockSpec((bm, f), lambda i: (i, 0)),
            pl.BlockSpec((bm, f), lambda i: (i, 0)),
            pl.BlockSpec((bm, 1), lambda i: (i, 0)),
            pl.BlockSpec((1, f), lambda i: (0, 0)),
        ],
        out_specs=pl.BlockSpec((bm, f), lambda i: (i, 0)),
        out_shape=jax.ShapeDtypeStruct((n, f), jnp.float32),
    )(acc, hp, deg_col, b.reshape(1, f))


def _score_body(x_ref, w_ref, o_ref, y_ref, *, n, bm):
    wv = w_ref[...]
    nrm = jnp.sqrt(jnp.sum(wv * wv))
    xv = x_ref[...]
    s = jnp.dot(xv, wv, preferred_element_type=jnp.float32) / nrm
    t = jnp.tanh(s)
    i = pl.program_id(0)
    rows = lax.broadcasted_iota(jnp.int32, t.shape, 0) + i * bm
    o_ref[...] = jnp.where(rows < n, t, -2.0)
    y_ref[...] = xv * t


def _score(x_pad, w, n, bm=ROW_ALIGN):
    npad, c = x_pad.shape
    return pl.pallas_call(
        functools.partial(_score_body, n=n, bm=bm),
        grid=(npad // bm,),
        in_specs=[pl.BlockSpec((bm, c), lambda i: (i, 0)),
                  pl.BlockSpec((c, 1), lambda i: (0, 0))],
        out_specs=[pl.BlockSpec((bm, 1), lambda i: (i, 0)),
                   pl.BlockSpec((bm, c), lambda i: (i, 0))],
        out_shape=[jax.ShapeDtypeStruct((npad, 1), jnp.float32),
                   jax.ShapeDtypeStruct((npad, c), jnp.float32)],
    )(x_pad, w.reshape(c, 1))


def _orderable_i32(b):
    return jnp.where(b < 0, ~b, b | jnp.int32(-2147483648))


def _thresh_body(s_ref, m_ref, *, k):
    b = lax.bitcast_convert_type(s_ref[...], jnp.int32)
    keys = _orderable_i32(b).astype(jnp.uint32)
    acc = jnp.uint32(0)
    for bitpos in range(31, -1, -1):
        cand = acc | jnp.uint32(1 << bitpos)
        cnt = jnp.sum((keys >= cand).astype(jnp.int32))
        acc = jnp.where(cnt >= k, cand, acc)
    m_ref[...] = (keys >= acc).astype(jnp.int32)


def _thresh(score_col, k):
    npad = score_col.shape[0]
    return pl.pallas_call(
        functools.partial(_thresh_body, k=k),
        in_specs=[pl.BlockSpec((npad, 1), lambda: (0, 0))],
        out_specs=pl.BlockSpec((npad, 1), lambda: (0, 0)),
        out_shape=jax.ShapeDtypeStruct((npad, 1), jnp.int32),
    )(score_col)


def _gather_body(table, idx, out, idxv, rows, sem, *, bg, nbat, trash):
    cid = lax.axis_index("c")
    sid = lax.axis_index("s")
    wid = cid + NUM_SC * sid

    def body(i, _):
        base = (wid + i * NUM_W) * bg
        pltpu.sync_copy(idx.at[pl.ds(base, bg)], idxv)
        for v in range(bg // 16):
            sl = pl.ds(v * 16, 16)
            m = idxv[sl]
            idxv[sl] = jnp.where(m < 0, trash, m)
        pltpu.async_copy(table.at[idxv], rows, sem).wait()
        pltpu.sync_copy(rows, out.at[pl.ds(base, bg)])
        return 0

    lax.fori_loop(0, nbat, body, 0)


def _gather_rows(table, idx, bg=16):
    nout = idx.shape[0]
    npad_in, f = table.shape
    nbat = nout // (NUM_W * bg)
    body = functools.partial(_gather_body, bg=bg, nbat=nbat,
                             trash=npad_in - 1)
    return pl.kernel(
        body,
        out_type=jax.ShapeDtypeStruct((nout, f), jnp.float32),
        mesh=_sc_mesh(),
        compiler_params=_SC_PARAMS,
        scratch_types=[
            pltpu.VMEM((bg,), jnp.int32),
            pltpu.VMEM((bg, f), jnp.float32),
            pltpu.SemaphoreType.DMA,
        ],
    )(table, idx)


def _relabel_body(mp, srcp, dstp, rs, rd, idxv, outv, sem, *, nbat_total):
    cid = lax.axis_index("c")
    sid = lax.axis_index("s")
    wid = cid + NUM_SC * sid
    nmine = (nbat_total - wid + NUM_W - 1) // NUM_W

    def body(i, _):
        base = (wid + i * NUM_W) * 128
        pltpu.sync_copy(srcp.at[pl.ds(base, 128)], idxv)
        pltpu.async_copy(mp.at[idxv], outv, sem).wait()
        pltpu.sync_copy(outv, rs.at[pl.ds(base, 128)])
        pltpu.sync_copy(dstp.at[pl.ds(base, 128)], idxv)
        pltpu.async_copy(mp.at[idxv], outv, sem).wait()
        pltpu.sync_copy(outv, rd.at[pl.ds(base, 128)])
        return 0

    lax.fori_loop(0, nmine, body, 0)


def _relabel(mapping, srcp, dstp):
    ne = srcp.shape[0]
    body = functools.partial(_relabel_body, nbat_total=ne // 128)
    return pl.kernel(
        body,
        out_type=(jax.ShapeDtypeStruct((ne,), jnp.int32),
                  jax.ShapeDtypeStruct((ne,), jnp.int32)),
        mesh=_sc_mesh(),
        compiler_params=_SC_PARAMS,
        scratch_types=[
            pltpu.VMEM((128,), jnp.int32),
            pltpu.VMEM((128,), jnp.int32),
            pltpu.SemaphoreType.DMA,
        ],
    )(mapping, srcp, dstp)


def _scatter_body(ysrc, idx, out, idxv, rows, *, bg, nbat, trash):
    cid = lax.axis_index("c")
    sid = lax.axis_index("s")
    wid = cid + NUM_SC * sid

    def body(i, _):
        base = (wid + i * NUM_W) * bg
        pltpu.sync_copy(idx.at[pl.ds(base, bg)], idxv)
        for v in range(bg // 16):
            sl = pl.ds(v * 16, 16)
            m = idxv[sl]
            idxv[sl] = jnp.where(m < 0, trash, m)
        pltpu.sync_copy(ysrc.at[pl.ds(base, bg)], rows)
        pltpu.sync_copy(rows, out.at[idxv])
        return 0

    lax.fori_loop(0, nbat, body, 0)


def _scatter_rows(ysrc, idx, npad_out, bg=16):
    npad_in, f = ysrc.shape
    nbat = npad_in // (NUM_W * bg)
    body = functools.partial(_scatter_body, bg=bg, nbat=nbat,
                             trash=npad_out - 1)
    return pl.kernel(
        body,
        out_type=jax.ShapeDtypeStruct((npad_out, f), jnp.float32),
        mesh=_sc_mesh(),
        compiler_params=_SC_PARAMS,
        scratch_types=[
            pltpu.VMEM((bg,), jnp.int32),
            pltpu.VMEM((bg, f), jnp.float32),
        ],
    )(ysrc, idx)


def _agg_body(hp, srcp, dstp, nbv, out, idx_s, idx_d, rows, nb_v, sem,
              acc_sp, *, npad, f, be, ecap):
    cid = lax.axis_index("c")
    sid = lax.axis_index("s")
    half = npad // NUM_SC

    def zrow(r, _):
        for c in range(f // 16):
            rows[r, pl.ds(c * 16, 16)] = jnp.zeros((16,), jnp.float32)
        return 0

    lax.fori_loop(0, 16, zrow, 0)
    rpt = half // NUM_TILES

    def zcp(j, _):
        pltpu.sync_copy(rows.at[pl.ds(0, 16)],
                        acc_sp.at[pl.ds(sid * rpt + j * 16, 16)])
        return 0

    lax.fori_loop(0, rpt // 16, zcp, 0)
    plsc.subcore_barrier()

    pltpu.sync_copy(nbv, nb_v)
    nbs = nb_v[...]
    nb = jnp.where(cid == 0, nbs[0], nbs[1])
    nmine = (nb - sid + NUM_TILES - 1) // NUM_TILES
    rebase = cid * half

    def body(i, _):
        base = cid * ecap + (sid + i * NUM_TILES) * be
        pltpu.sync_copy(srcp.at[pl.ds(base, be)], idx_s)
        pltpu.sync_copy(dstp.at[pl.ds(base, be)], idx_d)
        for v in range(be // 16):
            sl = pl.ds(v * 16, 16)
            idx_d[sl] = idx_d[sl] - rebase
        pltpu.async_copy(hp.at[idx_s], rows, sem).wait()
        pltpu.sync_copy(rows, acc_sp.at[idx_d], add=True)
        return 0

    lax.fori_loop(0, nmine, body, 0)
    plsc.subcore_barrier()

    def dump(j, _):
        r0 = sid * rpt + j * 16
        pltpu.sync_copy(acc_sp.at[pl.ds(r0, 16)], rows.at[pl.ds(0, 16)])
        pltpu.sync_copy(rows.at[pl.ds(0, 16)], out.at[cid, pl.ds(r0, 16)])
        return 0

    lax.fori_loop(0, rpt // 16, dump, 0)


def _agg(hp, srcp, dstp, nbv, npad, f, be):
    ecap = srcp.shape[0] // NUM_SC
    body = functools.partial(_agg_body, npad=npad, f=f, be=be, ecap=ecap)
    out = pl.kernel(
        body,
        out_type=jax.ShapeDtypeStruct((NUM_SC, npad // NUM_SC, f),
                                      jnp.float32),
        mesh=_sc_mesh(),
        compiler_params=_SC_PARAMS,
        scratch_types=[
            pltpu.VMEM((be,), jnp.int32),
            pltpu.VMEM((be,), jnp.int32),
            pltpu.VMEM((be, f), jnp.float32),
            pltpu.VMEM((16,), jnp.int32),
            pltpu.SemaphoreType.DMA,
            pltpu.VMEM_SHARED((npad // NUM_SC, f), jnp.float32),
        ],
    )(hp, srcp, dstp, nbv)
    return out.reshape(npad, f)


def _gcn(x_pad, w, b, deg_col, srcp, dstp, nbv, n, npad, be, relu):
    hp = _mm_prescale(x_pad, w, deg_col, n)
    acc = _agg(hp, srcp, dstp, nbv, npad, w.shape[1], be)
    return _combine(acc, hp, deg_col, b, n, relu)


def _be_for(f):
    return 64 if f >= 1024 else 128


def _pad_rows(a, npad):
    return jnp.pad(a, ((0, npad - a.shape[0]), (0, 0)))


def _bucket(src, dst, valid, e_cap, npad):
    half = npad // NUM_SC
    low = dst < half
    m0 = valid & low
    m1 = valid & ~low
    p0 = jnp.cumsum(m0.astype(jnp.int32)) - 1
    p1 = jnp.cumsum(m1.astype(jnp.int32)) - 1
    cnt0 = jnp.sum(m0.astype(jnp.int32))
    cnt1 = jnp.sum(m1.astype(jnp.int32))
    drop = 2 * e_cap
    tgt = jnp.where(m0, p0, jnp.where(m1, e_cap + p1, drop))
    bsrc = jnp.full((2 * e_cap,), npad - 1, jnp.int32).at[tgt].set(
        src, mode="drop")
    ar2 = jnp.arange(2 * e_cap, dtype=jnp.int32)
    fill = jnp.where(ar2 < e_cap, 0, half)
    bdst = fill.at[tgt].set(dst, mode="drop")
    return bsrc, bdst, cnt0, cnt1


def _nbv(cnt0, cnt1, be):
    return jnp.stack([(cnt0 + be - 1) // be, (cnt1 + be - 1) // be] +
                     [jnp.int32(0)] * 14).astype(jnp.int32)


def kernel(x, edge_index, Wd, bd, pw, Wu, bu):
    ns = [x.shape[0]]
    for _ in range(DEPTH):
        ns.append(int(math.ceil(RATIO * ns[-1])))
    npads = [_round_up(n + 1, ROW_ALIGN) for n in ns]
    e_cap = edge_index.shape[1]

    src = edge_index[0].astype(jnp.int32)
    dst = edge_index[1].astype(jnp.int32)

    def deg_of(bsrc, bdst, c0, c1, n, npad):
        onecol = _pad_rows(jnp.ones((n, 16), jnp.float32), npad)
        d16 = _agg(onecol, bsrc, bdst, _nbv(c0, c1, 128), npad, 16, 128)
        return d16[:, :1]

    n0, npad0 = ns[0], npads[0]
    bsrc, bdst, cnt0, cnt1 = _bucket(
        src, dst, jnp.ones((e_cap,), bool), e_cap, npad0)
    deg0_col = deg_of(bsrc, bdst, cnt0, cnt1, n0, npad0)
    be0 = _be_for(Wd[0].shape[1])
    xcur = _gcn(_pad_rows(x, npad0), Wd[0], bd[0], deg0_col,
                bsrc, bdst, _nbv(cnt0, cnt1, be0), n0, npad0, be0, relu=True)

    xs = [xcur]
    lvl = [(bsrc, bdst, cnt0, cnt1, deg0_col)]
    maps = []

    for i in range(1, DEPTH + 1):
        n_prev, n_i = ns[i - 1], ns[i]
        npad_prev, npad = npads[i - 1], npads[i]
        bsrc_p, bdst_p, c0_p, c1_p, _ = lvl[i - 1]

        x_prev_pad = _pad_rows(xcur, npad_prev)
        score_col, y = _score(x_prev_pad, pw[i - 1], n_prev)
        msel = _thresh(score_col, n_i).reshape(npad_prev)
        pos = jnp.cumsum(msel) - 1
        mapping = jnp.where((msel > 0) & (pos < n_i), pos, -1).astype(jnp.int32)

        rs, rd = _relabel(mapping, bsrc_p, bdst_p)
        valid = (rs >= 0) & (rd >= 0)

        fo = Wd[i].shape[1]
        be = _be_for(fo)
        bsrc, bdst, cnt0, cnt1 = _bucket(rs, rd, valid, e_cap, npad)
        deg_col = deg_of(bsrc, bdst, cnt0, cnt1, n_i, npad)
        xn_pad = _scatter_rows(y, mapping, npad)
        xcur = _gcn(xn_pad, Wd[i], bd[i], deg_col,
                    bsrc, bdst, _nbv(cnt0, cnt1, be), n_i, npad, be,
                    relu=True)
        maps.append(mapping)
        if i < DEPTH:
            xs.append(xcur)
            lvl.append((bsrc, bdst, cnt0, cnt1, deg_col))

    for i in range(DEPTH):
        j = DEPTH - 1 - i
        n_j, npad_j = ns[j], npads[j]
        res = xs[j]
        xcur_pad = _pad_rows(xcur, npads[j + 1])
        up_pad = _gather_rows(xcur_pad, maps[j])
        cat = jnp.concatenate([_pad_rows(res, npad_j), up_pad], axis=-1)

        fo = Wu[i].shape[1]
        be = _be_for(fo)
        bsrc_j, bdst_j, c0_j, c1_j, deg_j = lvl[j]
        xcur = _gcn(cat, Wu[i], bu[i], deg_j, bsrc_j, bdst_j,
                    _nbv(c0_j, c1_j, be), n_j, npad_j, be,
                    relu=(i < DEPTH - 1))

    return xcur


# speedup vs baseline: 3.3113x; 1.4627x over previous
import functools
import math

import jax
import jax.numpy as jnp
from jax import lax
from jax.experimental import pallas as pl
from jax.experimental.pallas import tpu as pltpu
from jax.experimental.pallas import tpu_sc as plsc

DEPTH = 5
RATIO = 0.5
NUM_SC = 2
NUM_TILES = 16
NUM_W = NUM_SC * NUM_TILES
ROW_ALIGN = 512


def _round_up(a, b):
    return (a + b - 1) // b * b


def _sc_mesh():
    return plsc.VectorSubcoreMesh(core_axis_name="c", subcore_axis_name="s",
                                  num_cores=NUM_SC, num_subcores=NUM_TILES)


_SC_PARAMS = pltpu.CompilerParams(use_tc_tiling_on_sc=False)


def _mm_body(a_ref, w_ref, deg_ref, o_ref, *, n, bm):
    acc = jnp.dot(a_ref[...], w_ref[...], preferred_element_type=jnp.float32)
    rs = 1.0 / jnp.sqrt(deg_ref[...] + 2.0)
    i = pl.program_id(0)
    rows = lax.broadcasted_iota(jnp.int32, acc.shape, 0) + i * bm
    o_ref[...] = jnp.where(rows < n, acc * rs, 0.0)


def _mm_prescale(a, w, deg_col, n, bm=ROW_ALIGN):
    mpad, k = a.shape
    f = w.shape[1]
    return pl.pallas_call(
        functools.partial(_mm_body, n=n, bm=bm),
        grid=(mpad // bm,),
        in_specs=[
            pl.BlockSpec((bm, k), lambda i: (i, 0)),
            pl.BlockSpec((k, f), lambda i: (0, 0)),
            pl.BlockSpec((bm, 1), lambda i: (i, 0)),
        ],
        out_specs=pl.BlockSpec((bm, f), lambda i: (i, 0)),
        out_shape=jax.ShapeDtypeStruct((mpad, f), jnp.float32),
    )(a, w, deg_col)


def _combine_body(acc_ref, hp_ref, deg_ref, b_ref, o_ref, *, relu):
    s = acc_ref[...] + 2.0 * hp_ref[...]
    rs = 1.0 / jnp.sqrt(deg_ref[...] + 2.0)
    r = s * rs + b_ref[...]
    if relu:
        r = jnp.maximum(r, 0.0)
    o_ref[...] = r


def _combine(acc, hp, deg_col, b, n, relu, bm=ROW_ALIGN):
    f = hp.shape[1]
    return pl.pallas_call(
        functools.partial(_combine_body, relu=relu),
        grid=(_round_up(n, bm) // bm,),
        in_specs=[
            pl.BlockSpec((bm, f), lambda i: (i, 0)),
            pl.BlockSpec((bm, f), lambda i: (i, 0)),
            pl.BlockSpec((bm, 1), lambda i: (i, 0)),
            pl.BlockSpec((1, f), lambda i: (0, 0)),
        ],
        out_specs=pl.BlockSpec((bm, f), lambda i: (i, 0)),
        out_shape=jax.ShapeDtypeStruct((n, f), jnp.float32),
    )(acc, hp, deg_col, b.reshape(1, f))


def _score_body(x_ref, w_ref, o_ref, y_ref, *, n, bm):
    wv = w_ref[...]
    nrm = jnp.sqrt(jnp.sum(wv * wv))
    xv = x_ref[...]
    s = jnp.dot(xv, wv, preferred_element_type=jnp.float32) / nrm
    t = jnp.tanh(s)
    i = pl.program_id(0)
    rows = lax.broadcasted_iota(jnp.int32, t.shape, 0) + i * bm
    o_ref[...] = jnp.where(rows < n, t, -2.0)
    y_ref[...] = xv * t


def _score(x_pad, w, n, bm=ROW_ALIGN):
    npad, c = x_pad.shape
    return pl.pallas_call(
        functools.partial(_score_body, n=n, bm=bm),
        grid=(npad // bm,),
        in_specs=[pl.BlockSpec((bm, c), lambda i: (i, 0)),
                  pl.BlockSpec((c, 1), lambda i: (0, 0))],
        out_specs=[pl.BlockSpec((bm, 1), lambda i: (i, 0)),
                   pl.BlockSpec((bm, c), lambda i: (i, 0))],
        out_shape=[jax.ShapeDtypeStruct((npad, 1), jnp.float32),
                   jax.ShapeDtypeStruct((npad, c), jnp.float32)],
    )(x_pad, w.reshape(c, 1))


def _orderable_i32(b):
    return jnp.where(b < 0, ~b, b | jnp.int32(-2147483648))


def _thresh_body(s_ref, m_ref, *, k):
    b = lax.bitcast_convert_type(s_ref[...], jnp.int32)
    keys = _orderable_i32(b).astype(jnp.uint32)
    acc = jnp.uint32(0)
    for bitpos in range(31, -1, -1):
        cand = acc | jnp.uint32(1 << bitpos)
        cnt = jnp.sum((keys >= cand).astype(jnp.int32))
        acc = jnp.where(cnt >= k, cand, acc)
    m_ref[...] = (keys >= acc).astype(jnp.int32)


def _thresh(score_col, k):
    npad = score_col.shape[0]
    return pl.pallas_call(
        functools.partial(_thresh_body, k=k),
        in_specs=[pl.BlockSpec((npad, 1), lambda: (0, 0))],
        out_specs=pl.BlockSpec((npad, 1), lambda: (0, 0)),
        out_shape=jax.ShapeDtypeStruct((npad, 1), jnp.int32),
    )(score_col)


def _gather_body(table, idx, out, idxv, rows, sem, *, bg, nbat, trash):
    cid = lax.axis_index("c")
    sid = lax.axis_index("s")
    wid = cid + NUM_SC * sid

    def body(i, _):
        base = (wid + i * NUM_W) * bg
        pltpu.sync_copy(idx.at[pl.ds(base, bg)], idxv)
        for v in range(bg // 16):
            sl = pl.ds(v * 16, 16)
            m = idxv[sl]
            idxv[sl] = jnp.where(m < 0, trash, m)
        pltpu.async_copy(table.at[idxv], rows, sem).wait()
        pltpu.sync_copy(rows, out.at[pl.ds(base, bg)])
        return 0

    lax.fori_loop(0, nbat, body, 0)


def _gather_rows(table, idx, bg=16):
    nout = idx.shape[0]
    npad_in, f = table.shape
    nbat = nout // (NUM_W * bg)
    body = functools.partial(_gather_body, bg=bg, nbat=nbat,
                             trash=npad_in - 1)
    return pl.kernel(
        body,
        out_type=jax.ShapeDtypeStruct((nout, f), jnp.float32),
        mesh=_sc_mesh(),
        compiler_params=_SC_PARAMS,
        scratch_types=[
            pltpu.VMEM((bg,), jnp.int32),
            pltpu.VMEM((bg, f), jnp.float32),
            pltpu.SemaphoreType.DMA,
        ],
    )(table, idx)


def _relabel_body(mp, srcp, dstp, nbv, rs, rd, idxv, outv, nb_v, sem, *,
                  ecap):
    cid = lax.axis_index("c")
    sid = lax.axis_index("s")
    pltpu.sync_copy(nbv, nb_v)
    nbs = nb_v[...]
    nb = jnp.where(cid == 0, nbs[0], nbs[1])
    nmine = (nb - sid + NUM_TILES - 1) // NUM_TILES

    def body(i, _):
        base = cid * ecap + (sid + i * NUM_TILES) * 128
        pltpu.sync_copy(srcp.at[pl.ds(base, 128)], idxv)
        pltpu.async_copy(mp.at[idxv], outv, sem).wait()
        pltpu.sync_copy(outv, rs.at[pl.ds(base, 128)])
        pltpu.sync_copy(dstp.at[pl.ds(base, 128)], idxv)
        pltpu.async_copy(mp.at[idxv], outv, sem).wait()
        pltpu.sync_copy(outv, rd.at[pl.ds(base, 128)])
        return 0

    lax.fori_loop(0, nmine, body, 0)


def _relabel(mapping, srcp, dstp, nbv):
    ne = srcp.shape[0]
    body = functools.partial(_relabel_body, ecap=ne // NUM_SC)
    return pl.kernel(
        body,
        out_type=(jax.ShapeDtypeStruct((ne,), jnp.int32),
                  jax.ShapeDtypeStruct((ne,), jnp.int32)),
        mesh=_sc_mesh(),
        compiler_params=_SC_PARAMS,
        scratch_types=[
            pltpu.VMEM((128,), jnp.int32),
            pltpu.VMEM((128,), jnp.int32),
            pltpu.VMEM((16,), jnp.int32),
            pltpu.SemaphoreType.DMA,
        ],
    )(mapping, srcp, dstp, nbv)


def _scatter_body(ysrc, idx, out, idxv, rows, *, bg, nbat, trash):
    cid = lax.axis_index("c")
    sid = lax.axis_index("s")
    wid = cid + NUM_SC * sid

    def body(i, _):
        base = (wid + i * NUM_W) * bg
        pltpu.sync_copy(idx.at[pl.ds(base, bg)], idxv)
        for v in range(bg // 16):
            sl = pl.ds(v * 16, 16)
            m = idxv[sl]
            idxv[sl] = jnp.where(m < 0, trash, m)
        pltpu.sync_copy(ysrc.at[pl.ds(base, bg)], rows)
        pltpu.sync_copy(rows, out.at[idxv])
        return 0

    lax.fori_loop(0, nbat, body, 0)


def _scatter_rows(ysrc, idx, npad_out, bg=16):
    npad_in, f = ysrc.shape
    nbat = npad_in // (NUM_W * bg)
    body = functools.partial(_scatter_body, bg=bg, nbat=nbat,
                             trash=npad_out - 1)
    return pl.kernel(
        body,
        out_type=jax.ShapeDtypeStruct((npad_out, f), jnp.float32),
        mesh=_sc_mesh(),
        compiler_params=_SC_PARAMS,
        scratch_types=[
            pltpu.VMEM((bg,), jnp.int32),
            pltpu.VMEM((bg, f), jnp.float32),
        ],
    )(ysrc, idx)


def _agg_body(hp, srcp, dstp, nbv, out, idx_s, idx_d, rows, nb_v, sem,
              acc_sp, *, npad, f, be, ecap):
    cid = lax.axis_index("c")
    sid = lax.axis_index("s")
    half = npad // NUM_SC

    def zrow(r, _):
        for c in range(f // 16):
            rows[r, pl.ds(c * 16, 16)] = jnp.zeros((16,), jnp.float32)
        return 0

    lax.fori_loop(0, 16, zrow, 0)
    rpt = half // NUM_TILES

    def zcp(j, _):
        pltpu.sync_copy(rows.at[pl.ds(0, 16)],
                        acc_sp.at[pl.ds(sid * rpt + j * 16, 16)])
        return 0

    lax.fori_loop(0, rpt // 16, zcp, 0)
    plsc.subcore_barrier()

    pltpu.sync_copy(nbv, nb_v)
    nbs = nb_v[...]
    nb = jnp.where(cid == 0, nbs[0], nbs[1])
    nmine = (nb - sid + NUM_TILES - 1) // NUM_TILES
    rebase = cid * half

    def body(i, _):
        base = cid * ecap + (sid + i * NUM_TILES) * be
        pltpu.sync_copy(srcp.at[pl.ds(base, be)], idx_s)
        pltpu.sync_copy(dstp.at[pl.ds(base, be)], idx_d)
        for v in range(be // 16):
            sl = pl.ds(v * 16, 16)
            idx_d[sl] = idx_d[sl] - rebase
        pltpu.async_copy(hp.at[idx_s], rows, sem).wait()
        pltpu.sync_copy(rows, acc_sp.at[idx_d], add=True)
        return 0

    lax.fori_loop(0, nmine, body, 0)
    plsc.subcore_barrier()

    def dump(j, _):
        r0 = sid * rpt + j * 16
        pltpu.sync_copy(acc_sp.at[pl.ds(r0, 16)], rows.at[pl.ds(0, 16)])
        pltpu.sync_copy(rows.at[pl.ds(0, 16)], out.at[cid, pl.ds(r0, 16)])
        return 0

    lax.fori_loop(0, rpt // 16, dump, 0)


def _agg(hp, srcp, dstp, nbv, npad, f, be):
    ecap = srcp.shape[0] // NUM_SC
    body = functools.partial(_agg_body, npad=npad, f=f, be=be, ecap=ecap)
    out = pl.kernel(
        body,
        out_type=jax.ShapeDtypeStruct((NUM_SC, npad // NUM_SC, f),
                                      jnp.float32),
        mesh=_sc_mesh(),
        compiler_params=_SC_PARAMS,
        scratch_types=[
            pltpu.VMEM((be,), jnp.int32),
            pltpu.VMEM((be,), jnp.int32),
            pltpu.VMEM((be, f), jnp.float32),
            pltpu.VMEM((16,), jnp.int32),
            pltpu.SemaphoreType.DMA,
            pltpu.VMEM_SHARED((npad // NUM_SC, f), jnp.float32),
        ],
    )(hp, srcp, dstp, nbv)
    return out.reshape(npad, f)


def _gcn(x_pad, w, b, deg_col, srcp, dstp, nbv, n, npad, be, relu):
    hp = _mm_prescale(x_pad, w, deg_col, n)
    acc = _agg(hp, srcp, dstp, nbv, npad, w.shape[1], be)
    return _combine(acc, hp, deg_col, b, n, relu)


def _be_for(f):
    return 64 if f >= 1024 else 128


def _pad_rows(a, npad):
    return jnp.pad(a, ((0, npad - a.shape[0]), (0, 0)))


def _bucket(src, dst, valid, e_cap, npad):
    half = npad // NUM_SC
    low = dst < half
    m0 = valid & low
    m1 = valid & ~low
    p0 = jnp.cumsum(m0.astype(jnp.int32)) - 1
    p1 = jnp.cumsum(m1.astype(jnp.int32)) - 1
    cnt0 = jnp.sum(m0.astype(jnp.int32))
    cnt1 = jnp.sum(m1.astype(jnp.int32))
    drop = 2 * e_cap
    tgt = jnp.where(m0, p0, jnp.where(m1, e_cap + p1, drop))
    bsrc = jnp.full((2 * e_cap,), npad - 1, jnp.int32).at[tgt].set(
        src, mode="drop")
    ar2 = jnp.arange(2 * e_cap, dtype=jnp.int32)
    fill = jnp.where(ar2 < e_cap, 0, half)
    bdst = fill.at[tgt].set(dst, mode="drop")
    return bsrc, bdst, cnt0, cnt1


def _nbv(cnt0, cnt1, be):
    return jnp.stack([(cnt0 + be - 1) // be, (cnt1 + be - 1) // be] +
                     [jnp.int32(0)] * 14).astype(jnp.int32)


def kernel(x, edge_index, Wd, bd, pw, Wu, bu):
    ns = [x.shape[0]]
    for _ in range(DEPTH):
        ns.append(int(math.ceil(RATIO * ns[-1])))
    npads = [_round_up(n + 1, ROW_ALIGN) for n in ns]
    e_cap = edge_index.shape[1]
    ar2 = jnp.arange(2 * e_cap, dtype=jnp.int32)

    def in_bucket(c0, c1):
        return jnp.where(ar2 < e_cap, ar2 < c0, (ar2 - e_cap) < c1)

    src = edge_index[0].astype(jnp.int32)
    dst = edge_index[1].astype(jnp.int32)

    def deg_of(bsrc, bdst, c0, c1, n, npad):
        onecol = _pad_rows(jnp.ones((n, 16), jnp.float32), npad)
        d16 = _agg(onecol, bsrc, bdst, _nbv(c0, c1, 128), npad, 16, 128)
        return d16[:, :1]

    n0, npad0 = ns[0], npads[0]
    bsrc, bdst, cnt0, cnt1 = _bucket(
        src, dst, jnp.ones((e_cap,), bool), e_cap, npad0)
    deg0_col = deg_of(bsrc, bdst, cnt0, cnt1, n0, npad0)
    be0 = _be_for(Wd[0].shape[1])
    xcur = _gcn(_pad_rows(x, npad0), Wd[0], bd[0], deg0_col,
                bsrc, bdst, _nbv(cnt0, cnt1, be0), n0, npad0, be0, relu=True)

    xs = [xcur]
    lvl = [(bsrc, bdst, cnt0, cnt1, deg0_col)]
    maps = []

    for i in range(1, DEPTH + 1):
        n_prev, n_i = ns[i - 1], ns[i]
        npad_prev, npad = npads[i - 1], npads[i]
        bsrc_p, bdst_p, c0_p, c1_p, _ = lvl[i - 1]

        x_prev_pad = _pad_rows(xcur, npad_prev)
        score_col, y = _score(x_prev_pad, pw[i - 1], n_prev)
        msel = _thresh(score_col, n_i).reshape(npad_prev)
        pos = jnp.cumsum(msel) - 1
        mapping = jnp.where((msel > 0) & (pos < n_i), pos, -1).astype(jnp.int32)

        rs, rd = _relabel(mapping, bsrc_p, bdst_p, _nbv(c0_p, c1_p, 128))
        valid = (rs >= 0) & (rd >= 0) & in_bucket(c0_p, c1_p)

        fo = Wd[i].shape[1]
        be = _be_for(fo)
        bsrc, bdst, cnt0, cnt1 = _bucket(rs, rd, valid, e_cap, npad)
        deg_col = deg_of(bsrc, bdst, cnt0, cnt1, n_i, npad)
        xn_pad = _scatter_rows(y, mapping, npad)
        xcur = _gcn(xn_pad, Wd[i], bd[i], deg_col,
                    bsrc, bdst, _nbv(cnt0, cnt1, be), n_i, npad, be,
                    relu=True)
        maps.append(mapping)
        if i < DEPTH:
            xs.append(xcur)
            lvl.append((bsrc, bdst, cnt0, cnt1, deg_col))

    for i in range(DEPTH):
        j = DEPTH - 1 - i
        n_j, npad_j = ns[j], npads[j]
        res = xs[j]
        xcur_pad = _pad_rows(xcur, npads[j + 1])
        up_pad = _gather_rows(xcur_pad, maps[j])
        cat = jnp.concatenate([_pad_rows(res, npad_j), up_pad], axis=-1)

        fo = Wu[i].shape[1]
        be = _be_for(fo)
        bsrc_j, bdst_j, c0_j, c1_j, deg_j = lvl[j]
        xcur = _gcn(cat, Wu[i], bu[i], deg_j, bsrc_j, bdst_j,
                    _nbv(c0_j, c1_j, be), n_j, npad_j, be,
                    relu=(i < DEPTH - 1))

    return xcur


# speedup vs baseline: 3.3267x; 1.0046x over previous
import functools
import math

import jax
import jax.numpy as jnp
from jax import lax
from jax.experimental import pallas as pl
from jax.experimental.pallas import tpu as pltpu
from jax.experimental.pallas import tpu_sc as plsc

DEPTH = 5
RATIO = 0.5
NUM_SC = 2
NUM_TILES = 16
NUM_W = NUM_SC * NUM_TILES
ROW_ALIGN = 512


def _round_up(a, b):
    return (a + b - 1) // b * b


def _sc_mesh():
    return plsc.VectorSubcoreMesh(core_axis_name="c", subcore_axis_name="s",
                                  num_cores=NUM_SC, num_subcores=NUM_TILES)


_SC_PARAMS = pltpu.CompilerParams(use_tc_tiling_on_sc=False)


def _mm_body(a_ref, w_ref, deg_ref, o_ref, *, n, bm):
    acc = jnp.dot(a_ref[...], w_ref[...], preferred_element_type=jnp.float32)
    rs = 1.0 / jnp.sqrt(deg_ref[...] + 2.0)
    i = pl.program_id(0)
    rows = lax.broadcasted_iota(jnp.int32, acc.shape, 0) + i * bm
    o_ref[...] = jnp.where(rows < n, acc * rs, 0.0)


def _mm_prescale(a, w, deg_col, n, bm=ROW_ALIGN):
    mpad, k = a.shape
    f = w.shape[1]
    return pl.pallas_call(
        functools.partial(_mm_body, n=n, bm=bm),
        grid=(mpad // bm,),
        in_specs=[
            pl.BlockSpec((bm, k), lambda i: (i, 0)),
            pl.BlockSpec((k, f), lambda i: (0, 0)),
            pl.BlockSpec((bm, 1), lambda i: (i, 0)),
        ],
        out_specs=pl.BlockSpec((bm, f), lambda i: (i, 0)),
        out_shape=jax.ShapeDtypeStruct((mpad, f), jnp.float32),
    )(a, w, deg_col)


def _combine_body(acc_ref, hp_ref, deg_ref, b_ref, o_ref, *, relu):
    s = acc_ref[...] + 2.0 * hp_ref[...]
    rs = 1.0 / jnp.sqrt(deg_ref[...] + 2.0)
    r = s * rs + b_ref[...]
    if relu:
        r = jnp.maximum(r, 0.0)
    o_ref[...] = r


def _combine(acc, hp, deg_col, b, n, relu, bm=ROW_ALIGN):
    f = hp.shape[1]
    return pl.pallas_call(
        functools.partial(_combine_body, relu=relu),
        grid=(_round_up(n, bm) // bm,),
        in_specs=[
            pl.BlockSpec((bm, f), lambda i: (i, 0)),
            pl.BlockSpec((bm, f), lambda i: (i, 0)),
            pl.BlockSpec((bm, 1), lambda i: (i, 0)),
            pl.BlockSpec((1, f), lambda i: (0, 0)),
        ],
        out_specs=pl.BlockSpec((bm, f), lambda i: (i, 0)),
        out_shape=jax.ShapeDtypeStruct((n, f), jnp.float32),
    )(acc, hp, deg_col, b.reshape(1, f))


def _score_body(x_ref, w_ref, o_ref, y_ref, *, n, bm):
    wv = w_ref[...]
    nrm = jnp.sqrt(jnp.sum(wv * wv))
    xv = x_ref[...]
    s = jnp.dot(xv, wv, preferred_element_type=jnp.float32) / nrm
    t = jnp.tanh(s)
    i = pl.program_id(0)
    rows = lax.broadcasted_iota(jnp.int32, t.shape, 0) + i * bm
    o_ref[...] = jnp.where(rows < n, t, -2.0)
    y_ref[...] = xv * t


def _score(x_pad, w, n, bm=ROW_ALIGN):
    npad, c = x_pad.shape
    return pl.pallas_call(
        functools.partial(_score_body, n=n, bm=bm),
        grid=(npad // bm,),
        in_specs=[pl.BlockSpec((bm, c), lambda i: (i, 0)),
                  pl.BlockSpec((c, 1), lambda i: (0, 0))],
        out_specs=[pl.BlockSpec((bm, 1), lambda i: (i, 0)),
                   pl.BlockSpec((bm, c), lambda i: (i, 0))],
        out_shape=[jax.ShapeDtypeStruct((npad, 1), jnp.float32),
                   jax.ShapeDtypeStruct((npad, c), jnp.float32)],
    )(x_pad, w.reshape(c, 1))


def _orderable_i32(b):
    return jnp.where(b < 0, ~b, b | jnp.int32(-2147483648))


def _thresh_body(s_ref, m_ref, *, k):
    b = lax.bitcast_convert_type(s_ref[...], jnp.int32)
    keys = _orderable_i32(b).astype(jnp.uint32)
    acc = jnp.uint32(0)
    for bitpos in range(31, -1, -1):
        cand = acc | jnp.uint32(1 << bitpos)
        cnt = jnp.sum((keys >= cand).astype(jnp.int32))
        acc = jnp.where(cnt >= k, cand, acc)
    m_ref[...] = (keys >= acc).astype(jnp.int32)


def _thresh(score_col, k):
    npad = score_col.shape[0]
    return pl.pallas_call(
        functools.partial(_thresh_body, k=k),
        in_specs=[pl.BlockSpec((npad, 1), lambda: (0, 0))],
        out_specs=pl.BlockSpec((npad, 1), lambda: (0, 0)),
        out_shape=jax.ShapeDtypeStruct((npad, 1), jnp.int32),
    )(score_col)


def _gather_body(table, idx, out, idxv, rows, sem, *, bg, nbat, trash):
    cid = lax.axis_index("c")
    sid = lax.axis_index("s")
    wid = cid + NUM_SC * sid

    def body(i, _):
        base = (wid + i * NUM_W) * bg
        pltpu.sync_copy(idx.at[pl.ds(base, bg)], idxv)
        for v in range(bg // 16):
            sl = pl.ds(v * 16, 16)
            m = idxv[sl]
            idxv[sl] = jnp.where(m < 0, trash, m)
        pltpu.async_copy(table.at[idxv], rows, sem).wait()
        pltpu.sync_copy(rows, out.at[pl.ds(base, bg)])
        return 0

    lax.fori_loop(0, nbat, body, 0)


def _gather_rows(table, idx, bg=16):
    nout = idx.shape[0]
    npad_in, f = table.shape
    nbat = nout // (NUM_W * bg)
    body = functools.partial(_gather_body, bg=bg, nbat=nbat,
                             trash=npad_in - 1)
    return pl.kernel(
        body,
        out_type=jax.ShapeDtypeStruct((nout, f), jnp.float32),
        mesh=_sc_mesh(),
        compiler_params=_SC_PARAMS,
        scratch_types=[
            pltpu.VMEM((bg,), jnp.int32),
            pltpu.VMEM((bg, f), jnp.float32),
            pltpu.SemaphoreType.DMA,
        ],
    )(table, idx)


def _relabel_body(mp, srcp, dstp, nbv, rs, rd, idxv, idxv2, outv, outv2,
                  nb_v, sem, sem2, *, ecap):
    cid = lax.axis_index("c")
    sid = lax.axis_index("s")
    pltpu.sync_copy(nbv, nb_v)
    nbs = nb_v[...]
    nb = jnp.where(cid == 0, nbs[0], nbs[1])
    nmine = (nb - sid + NUM_TILES - 1) // NUM_TILES

    def body(i, _):
        base = cid * ecap + (sid + i * NUM_TILES) * 128
        c1 = pltpu.async_copy(srcp.at[pl.ds(base, 128)], idxv, sem)
        c2 = pltpu.async_copy(dstp.at[pl.ds(base, 128)], idxv2, sem2)
        c1.wait()
        c2.wait()
        g1 = pltpu.async_copy(mp.at[idxv], outv, sem)
        g2 = pltpu.async_copy(mp.at[idxv2], outv2, sem2)
        g1.wait()
        g2.wait()
        s1 = pltpu.async_copy(outv, rs.at[pl.ds(base, 128)], sem)
        s2 = pltpu.async_copy(outv2, rd.at[pl.ds(base, 128)], sem2)
        s1.wait()
        s2.wait()
        return 0

    lax.fori_loop(0, nmine, body, 0)


def _relabel(mapping, srcp, dstp, nbv):
    ne = srcp.shape[0]
    body = functools.partial(_relabel_body, ecap=ne // NUM_SC)
    return pl.kernel(
        body,
        out_type=(jax.ShapeDtypeStruct((ne,), jnp.int32),
                  jax.ShapeDtypeStruct((ne,), jnp.int32)),
        mesh=_sc_mesh(),
        compiler_params=_SC_PARAMS,
        scratch_types=[
            pltpu.VMEM((128,), jnp.int32),
            pltpu.VMEM((128,), jnp.int32),
            pltpu.VMEM((128,), jnp.int32),
            pltpu.VMEM((128,), jnp.int32),
            pltpu.VMEM((16,), jnp.int32),
            pltpu.SemaphoreType.DMA,
            pltpu.SemaphoreType.DMA,
        ],
    )(mapping, srcp, dstp, nbv)


def _scatter_body(ysrc, idx, out, idxv, rows, *, bg, nbat, trash):
    cid = lax.axis_index("c")
    sid = lax.axis_index("s")
    wid = cid + NUM_SC * sid

    def body(i, _):
        base = (wid + i * NUM_W) * bg
        pltpu.sync_copy(idx.at[pl.ds(base, bg)], idxv)
        for v in range(bg // 16):
            sl = pl.ds(v * 16, 16)
            m = idxv[sl]
            idxv[sl] = jnp.where(m < 0, trash, m)
        pltpu.sync_copy(ysrc.at[pl.ds(base, bg)], rows)
        pltpu.sync_copy(rows, out.at[idxv])
        return 0

    lax.fori_loop(0, nbat, body, 0)


def _scatter_rows(ysrc, idx, npad_out, bg=16):
    npad_in, f = ysrc.shape
    nbat = npad_in // (NUM_W * bg)
    body = functools.partial(_scatter_body, bg=bg, nbat=nbat,
                             trash=npad_out - 1)
    return pl.kernel(
        body,
        out_type=jax.ShapeDtypeStruct((npad_out, f), jnp.float32),
        mesh=_sc_mesh(),
        compiler_params=_SC_PARAMS,
        scratch_types=[
            pltpu.VMEM((bg,), jnp.int32),
            pltpu.VMEM((bg, f), jnp.float32),
        ],
    )(ysrc, idx)


def _agg_body(hp, srcp, dstp, nbv, out, idx_s, idx_d, rows, nb_v, sem,
              acc_sp, *, npad, f, be, ecap):
    cid = lax.axis_index("c")
    sid = lax.axis_index("s")
    half = npad // NUM_SC

    def zrow(r, _):
        for c in range(f // 16):
            rows[r, pl.ds(c * 16, 16)] = jnp.zeros((16,), jnp.float32)
        return 0

    lax.fori_loop(0, 16, zrow, 0)
    rpt = half // NUM_TILES

    def zcp(j, _):
        pltpu.sync_copy(rows.at[pl.ds(0, 16)],
                        acc_sp.at[pl.ds(sid * rpt + j * 16, 16)])
        return 0

    lax.fori_loop(0, rpt // 16, zcp, 0)
    plsc.subcore_barrier()

    pltpu.sync_copy(nbv, nb_v)
    nbs = nb_v[...]
    nb = jnp.where(cid == 0, nbs[0], nbs[1])
    nmine = (nb - sid + NUM_TILES - 1) // NUM_TILES
    rebase = cid * half

    def body(i, _):
        base = cid * ecap + (sid + i * NUM_TILES) * be
        pltpu.sync_copy(srcp.at[pl.ds(base, be)], idx_s)
        pltpu.sync_copy(dstp.at[pl.ds(base, be)], idx_d)
        for v in range(be // 16):
            sl = pl.ds(v * 16, 16)
            idx_d[sl] = idx_d[sl] - rebase
        pltpu.async_copy(hp.at[idx_s], rows, sem).wait()
        pltpu.sync_copy(rows, acc_sp.at[idx_d], add=True)
        return 0

    lax.fori_loop(0, nmine, body, 0)
    plsc.subcore_barrier()

    def dump(j, _):
        r0 = sid * rpt + j * 16
        pltpu.sync_copy(acc_sp.at[pl.ds(r0, 16)], rows.at[pl.ds(0, 16)])
        pltpu.sync_copy(rows.at[pl.ds(0, 16)], out.at[cid, pl.ds(r0, 16)])
        return 0

    lax.fori_loop(0, rpt // 16, dump, 0)


def _agg(hp, srcp, dstp, nbv, npad, f, be):
    ecap = srcp.shape[0] // NUM_SC
    body = functools.partial(_agg_body, npad=npad, f=f, be=be, ecap=ecap)
    out = pl.kernel(
        body,
        out_type=jax.ShapeDtypeStruct((NUM_SC, npad // NUM_SC, f),
                                      jnp.float32),
        mesh=_sc_mesh(),
        compiler_params=_SC_PARAMS,
        scratch_types=[
            pltpu.VMEM((be,), jnp.int32),
            pltpu.VMEM((be,), jnp.int32),
            pltpu.VMEM((be, f), jnp.float32),
            pltpu.VMEM((16,), jnp.int32),
            pltpu.SemaphoreType.DMA,
            pltpu.VMEM_SHARED((npad // NUM_SC, f), jnp.float32),
        ],
    )(hp, srcp, dstp, nbv)
    return out.reshape(npad, f)


def _gcn(x_pad, w, b, deg_col, srcp, dstp, nbv, n, npad, be, relu):
    hp = _mm_prescale(x_pad, w, deg_col, n)
    acc = _agg(hp, srcp, dstp, nbv, npad, w.shape[1], be)
    return _combine(acc, hp, deg_col, b, n, relu)


def _be_for(f):
    return 64 if f >= 1024 else 128


def _pad_rows(a, npad):
    return jnp.pad(a, ((0, npad - a.shape[0]), (0, 0)))


def _bucket(src, dst, valid, e_cap, npad):
    half = npad // NUM_SC
    low = dst < half
    m0 = valid & low
    m1 = valid & ~low
    p0 = jnp.cumsum(m0.astype(jnp.int32)) - 1
    p1 = jnp.cumsum(m1.astype(jnp.int32)) - 1
    cnt0 = jnp.sum(m0.astype(jnp.int32))
    cnt1 = jnp.sum(m1.astype(jnp.int32))
    drop = 2 * e_cap
    tgt = jnp.where(m0, p0, jnp.where(m1, e_cap + p1, drop))
    bsrc = jnp.full((2 * e_cap,), npad - 1, jnp.int32).at[tgt].set(
        src, mode="drop")
    ar2 = jnp.arange(2 * e_cap, dtype=jnp.int32)
    fill = jnp.where(ar2 < e_cap, 0, half)
    bdst = fill.at[tgt].set(dst, mode="drop")
    return bsrc, bdst, cnt0, cnt1


def _nbv(cnt0, cnt1, be):
    return jnp.stack([(cnt0 + be - 1) // be, (cnt1 + be - 1) // be] +
                     [jnp.int32(0)] * 14).astype(jnp.int32)


def kernel(x, edge_index, Wd, bd, pw, Wu, bu):
    ns = [x.shape[0]]
    for _ in range(DEPTH):
        ns.append(int(math.ceil(RATIO * ns[-1])))
    npads = [_round_up(n + 1, ROW_ALIGN) for n in ns]
    e_cap = edge_index.shape[1]
    ar2 = jnp.arange(2 * e_cap, dtype=jnp.int32)

    def in_bucket(c0, c1):
        return jnp.where(ar2 < e_cap, ar2 < c0, (ar2 - e_cap) < c1)

    src = edge_index[0].astype(jnp.int32)
    dst = edge_index[1].astype(jnp.int32)

    def deg_of(bsrc, bdst, c0, c1, n, npad):
        onecol = _pad_rows(jnp.ones((n, 16), jnp.float32), npad)
        d16 = _agg(onecol, bsrc, bdst, _nbv(c0, c1, 128), npad, 16, 128)
        return d16[:, :1]

    n0, npad0 = ns[0], npads[0]
    bsrc, bdst, cnt0, cnt1 = _bucket(
        src, dst, jnp.ones((e_cap,), bool), e_cap, npad0)
    deg0_col = deg_of(bsrc, bdst, cnt0, cnt1, n0, npad0)
    be0 = _be_for(Wd[0].shape[1])
    xcur = _gcn(_pad_rows(x, npad0), Wd[0], bd[0], deg0_col,
                bsrc, bdst, _nbv(cnt0, cnt1, be0), n0, npad0, be0, relu=True)

    xs = [xcur]
    lvl = [(bsrc, bdst, cnt0, cnt1, deg0_col)]
    maps = []

    for i in range(1, DEPTH + 1):
        n_prev, n_i = ns[i - 1], ns[i]
        npad_prev, npad = npads[i - 1], npads[i]
        bsrc_p, bdst_p, c0_p, c1_p, _ = lvl[i - 1]

        x_prev_pad = _pad_rows(xcur, npad_prev)
        score_col, y = _score(x_prev_pad, pw[i - 1], n_prev)
        msel = _thresh(score_col, n_i).reshape(npad_prev)
        pos = jnp.cumsum(msel) - 1
        mapping = jnp.where((msel > 0) & (pos < n_i), pos, -1).astype(jnp.int32)

        rs, rd = _relabel(mapping, bsrc_p, bdst_p, _nbv(c0_p, c1_p, 128))
        valid = (rs >= 0) & (rd >= 0) & in_bucket(c0_p, c1_p)

        fo = Wd[i].shape[1]
        be = _be_for(fo)
        bsrc, bdst, cnt0, cnt1 = _bucket(rs, rd, valid, e_cap, npad)
        deg_col = deg_of(bsrc, bdst, cnt0, cnt1, n_i, npad)
        xn_pad = _scatter_rows(y, mapping, npad)
        xcur = _gcn(xn_pad, Wd[i], bd[i], deg_col,
                    bsrc, bdst, _nbv(cnt0, cnt1, be), n_i, npad, be,
                    relu=True)
        maps.append(mapping)
        if i < DEPTH:
            xs.append(xcur)
            lvl.append((bsrc, bdst, cnt0, cnt1, deg_col))

    for i in range(DEPTH):
        j = DEPTH - 1 - i
        n_j, npad_j = ns[j], npads[j]
        res = xs[j]
        xcur_pad = _pad_rows(xcur, npads[j + 1])
        up_pad = _gather_rows(xcur_pad, maps[j])
        cat = jnp.concatenate([_pad_rows(res, npad_j), up_pad], axis=-1)

        fo = Wu[i].shape[1]
        be = _be_for(fo)
        bsrc_j, bdst_j, c0_j, c1_j, deg_j = lvl[j]
        xcur = _gcn(cat, Wu[i], bu[i], deg_j, bsrc_j, bdst_j,
                    _nbv(c0_j, c1_j, be), n_j, npad_j, be,
                    relu=(i < DEPTH - 1))

    return xcur
